# Initial kernel scaffold; baseline (speedup 1.0000x reference)
import jax
import jax.numpy as jnp
from jax.experimental import pallas as pl


def kernel(nodes_u, nodes_v, history_u, history_ur, history_v, history_vr, social_adj, embed_u, embed_i, embed_r, gu1_w, gu1_b, gu2_w, gu2_b, ai1_w, ai1_b, ai2_w, ai2_b, ai3_w, ai3_b, gv1_w, gv1_b, gv2_w, gv2_b, aI1_w, aI1_b, aI2_w, aI2_b, aI3_w, aI3_b, aS1_w, aS1_b, aS2_w, aS2_b, aS3_w, aS3_b, um1_w, um1_b, um2_w, um2_b, ur1_w, ur1_b, ur2_w, ur2_b, vr1_w, vr1_b, vr2_w, vr2_b, uv1_w, uv1_b, uv2_w, uv2_b, uv3_w, uv3_b, bn1_g, bn1_b, bn1_m, bn1_v, bn2_g, bn2_b, bn2_m, bn2_v, bn3_g, bn3_b, bn3_m, bn3_v, bn4_g, bn4_b, bn4_m, bn4_v):
    raise NotImplementedError("write your pallas kernel here")


# speedup vs baseline: 2.8807x; 2.8807x over previous
import jax
import jax.numpy as jnp
from jax import lax
from jax.experimental import pallas as pl
from jax.experimental.pallas import tpu as pltpu
from jax.experimental.pallas import tpu_sc as plsc

B = 1024
L = 50
D = 64
NR = 5
NW = 32
BIG = B * L
BPW = BIG // NW
SPW = B // NW
BB = 128
G = B // BB

_f32 = jnp.float32


def _sc_gather(hist_v, soc, hist_u, nodes_u, nodes_v, embed_u, embed_i):
  mesh = plsc.VectorSubcoreMesh(core_axis_name="c", subcore_axis_name="s")
  out_type = (
      jax.ShapeDtypeStruct((BIG, D), _f32),
      jax.ShapeDtypeStruct((BIG, D), _f32),
      jax.ShapeDtypeStruct((BIG, D), _f32),
      jax.ShapeDtypeStruct((B, D), _f32),
      jax.ShapeDtypeStruct((B, D), _f32),
  )

  def body(hist_v, soc, hist_u, nodes_u, nodes_v, embed_u, embed_i,
           pt_out, un_out, qa_out, piu_out, qj_out,
           idx_v, rows_v, idx_s, rows_s, sem):
    wid = lax.axis_index("s") * 2 + lax.axis_index("c")
    base = wid * BPW
    for idx_hbm, table, out in ((hist_v, embed_u, pt_out),
                                (soc, embed_u, un_out),
                                (hist_u, embed_i, qa_out)):
      pltpu.sync_copy(idx_hbm.at[pl.ds(base, BPW)], idx_v)
      pltpu.async_copy(table.at[idx_v], rows_v, sem).wait()
      pltpu.sync_copy(rows_v, out.at[pl.ds(base, BPW)])
    sbase = wid * SPW
    for idx_hbm, table, out in ((nodes_u, embed_u, piu_out),
                                (nodes_v, embed_i, qj_out)):
      pltpu.sync_copy(idx_hbm.at[pl.ds(sbase, SPW)], idx_s)
      pltpu.async_copy(table.at[idx_s], rows_s, sem).wait()
      pltpu.sync_copy(rows_s, out.at[pl.ds(sbase, SPW)])

  return pl.kernel(
      body,
      out_type=out_type,
      mesh=mesh,
      scratch_types=[
          pltpu.VMEM((BPW,), jnp.int32),
          pltpu.VMEM((BPW, D), _f32),
          pltpu.VMEM((SPW,), jnp.int32),
          pltpu.VMEM((SPW, D), _f32),
          pltpu.SemaphoreType.DMA,
      ],
      compiler_params=pltpu.CompilerParams(use_tc_tiling_on_sc=False),
  )(hist_v, soc, hist_u, nodes_u, nodes_v, embed_u, embed_i)


def _tc_body(pt_ref, qa_ref, un_ref, qj_ref, piu_ref, vr_ref, ur_ref, erp_ref,
             gu1a, gu1b, gu1bias, gu2w, gu2b,
             ai1a, ai1b, ai1bias, ai2w, ai2b, ai3wT,
             gv1a, gv1b, gv1bias, gv2w, gv2b,
             aI1a, aI1b, aI1bias, aI2w, aI2b, aI3wT,
             aS1a, aS1b, aS1bias, aS2w, aS2b, aS3wT,
             um1a, um1b, um1bias, um2w, um2b,
             ur1w, ur1b, ur2w, ur2b,
             vr1w, vr1b, vr2w, vr2b,
             uv1a, uv1b, uv1bias, uv2w, uv2b, uv3wT, uv3b,
             s1, t1, s2, t2, s3, t3, s4, t4,
             out_ref):
  def mm(x, w):
    return lax.dot_general(x, w, (((1,), (0,)), ((), ())),
                           preferred_element_type=_f32)

  relu = lambda x: jnp.maximum(x, 0.0)
  erp = erp_ref[...]

  def nmlp(x, ids_ref, w1a, w1b, b1, w2, b2):
    r = mm(erp, w1b[...])
    ids = ids_ref[...]
    oh = (ids == lax.broadcasted_iota(jnp.int32, (BB * L, 8), 1)).astype(_f32)
    h = relu(mm(x, w1a[...]) + mm(oh, r) + b1[...])
    return relu(mm(h, w2[...]) + b2[...])

  def att_pool(neigh, urep, a1a, a1b, a1bias, a2w, a2b, a3wT):
    h = mm(neigh, a1a[...])
    hu = mm(urep, a1b[...]) + a1bias[...]
    h3 = relu(h.reshape(BB, L, D) + hu[:, None, :])
    h2 = relu(mm(h3.reshape(BB * L, D), a2w[...]) + a2b[...])
    logits = jnp.sum(h2.reshape(BB, L, D) * a3wT[...].reshape(1, 1, D), axis=2)
    m = jnp.max(logits, axis=1, keepdims=True)
    e = jnp.exp(logits - m)
    w = e / jnp.sum(e, axis=1, keepdims=True)
    return jnp.sum(neigh.reshape(BB, L, D) * w[:, :, None], axis=1)

  pt = pt_ref[...]
  qa = qa_ref[...]
  un = un_ref[...]
  qj = qj_ref[...]
  piu = piu_ref[...]

  fjt = nmlp(pt, vr_ref, gu1a, gu1b, gu1bias, gu2w, gu2b)
  zj = att_pool(fjt, qj, ai1a, ai1b, ai1bias, ai2w, ai2b, ai3wT)
  xia = nmlp(qa, ur_ref, gv1a, gv1b, gv1bias, gv2w, gv2b)
  hi_I = att_pool(xia, piu, aI1a, aI1b, aI1bias, aI2w, aI2b, aI3wT)
  hi_S = att_pool(un, piu, aS1a, aS1b, aS1bias, aS2w, aS2b, aS3wT)
  hi = relu(mm(hi_I, um1a[...]) + mm(hi_S, um1b[...]) + um1bias[...])
  hi = relu(mm(hi, um2w[...]) + um2b[...])
  hi = relu((mm(hi, ur1w[...]) + ur1b[...]) * s1[...] + t1[...])
  hi = mm(hi, ur2w[...]) + ur2b[...]
  zj = relu((mm(zj, vr1w[...]) + vr1b[...]) * s2[...] + t2[...])
  zj = relu(mm(zj, vr1w[...]) + vr1b[...])
  zj = mm(zj, vr2w[...]) + vr2b[...]
  x = relu((mm(hi, uv1a[...]) + mm(zj, uv1b[...]) + uv1bias[...]) * s3[...]
           + t3[...])
  x = relu((mm(x, uv2w[...]) + uv2b[...]) * s4[...] + t4[...])
  out_ref[...] = (jnp.sum(x * uv3wT[...], axis=1, keepdims=True) + uv3b[...])


def kernel(nodes_u, nodes_v, history_u, history_ur, history_v, history_vr,
           social_adj, embed_u, embed_i, embed_r,
           gu1_w, gu1_b, gu2_w, gu2_b, ai1_w, ai1_b, ai2_w, ai2_b, ai3_w,
           ai3_b, gv1_w, gv1_b, gv2_w, gv2_b, aI1_w, aI1_b, aI2_w, aI2_b,
           aI3_w, aI3_b, aS1_w, aS1_b, aS2_w, aS2_b, aS3_w, aS3_b, um1_w,
           um1_b, um2_w, um2_b, ur1_w, ur1_b, ur2_w, ur2_b, vr1_w, vr1_b,
           vr2_w, vr2_b, uv1_w, uv1_b, uv2_w, uv2_b, uv3_w, uv3_b,
           bn1_g, bn1_b, bn1_m, bn1_v, bn2_g, bn2_b, bn2_m, bn2_v,
           bn3_g, bn3_b, bn3_m, bn3_v, bn4_g, bn4_b, bn4_m, bn4_v):
  i32 = jnp.int32
  hist_v = history_v.astype(i32).reshape(BIG)
  soc = social_adj.astype(i32).reshape(BIG)
  hist_u = history_u.astype(i32).reshape(BIG)
  nu = nodes_u.astype(i32)
  nv = nodes_v.astype(i32)
  vr_ids = history_vr.astype(i32).reshape(BIG, 1)
  ur_ids = history_ur.astype(i32).reshape(BIG, 1)

  pt, un, qa, piu, qj = _sc_gather(hist_v, soc, hist_u, nu, nv,
                                   embed_u, embed_i)

  erp = jnp.pad(embed_r, ((0, 8 - NR), (0, 0)))

  def bn_fold(g, b, m, v):
    s = (g * lax.rsqrt(v + 1e-5)).reshape(1, -1)
    t = (b - m * g * lax.rsqrt(v + 1e-5)).reshape(1, -1)
    return s, t

  s1, t1 = bn_fold(bn1_g, bn1_b, bn1_m, bn1_v)
  s2, t2 = bn_fold(bn2_g, bn2_b, bn2_m, bn2_v)
  s3, t3 = bn_fold(bn3_g, bn3_b, bn3_m, bn3_v)
  s4, t4 = bn_fold(bn4_g, bn4_b, bn4_m, bn4_v)

  row = lambda b: b.reshape(1, -1)

  weights = [
      erp,
      gu1_w[:D], gu1_w[D:], row(gu1_b), gu2_w, row(gu2_b),
      ai1_w[:D], ai1_w[D:], row(ai1_b), ai2_w, row(ai2_b), ai3_w.reshape(1, D),
      gv1_w[:D], gv1_w[D:], row(gv1_b), gv2_w, row(gv2_b),
      aI1_w[:D], aI1_w[D:], row(aI1_b), aI2_w, row(aI2_b), aI3_w.reshape(1, D),
      aS1_w[:D], aS1_w[D:], row(aS1_b), aS2_w, row(aS2_b), aS3_w.reshape(1, D),
      um1_w[:D], um1_w[D:], row(um1_b), um2_w, row(um2_b),
      ur1_w, row(ur1_b), ur2_w, row(ur2_b),
      vr1_w, row(vr1_b), vr2_w, row(vr2_b),
      uv1_w[:D], uv1_w[D:], row(uv1_b), uv2_w, row(uv2_b),
      uv3_w.reshape(1, 16), uv3_b.reshape(1, 1),
      s1, t1, s2, t2, s3, t3, s4, t4,
  ]

  big_spec = pl.BlockSpec((BB * L, D), lambda g: (g, 0))
  small_spec = pl.BlockSpec((BB, D), lambda g: (g, 0))
  ids_spec = pl.BlockSpec((BB * L, 1), lambda g: (g, 0))
  wspec = lambda a: pl.BlockSpec(a.shape, lambda g, _n=len(a.shape): (0,) * _n)

  in_specs = ([big_spec, big_spec, big_spec, small_spec, small_spec,
               ids_spec, ids_spec]
              + [wspec(w) for w in weights])

  scores = pl.pallas_call(
      _tc_body,
      grid=(G,),
      in_specs=in_specs,
      out_specs=pl.BlockSpec((BB, 1), lambda g: (g, 0)),
      out_shape=jax.ShapeDtypeStruct((B, 1), _f32),
      compiler_params=pltpu.CompilerParams(
          dimension_semantics=("arbitrary",)),
  )(pt, qa, un, qj, piu, vr_ids, ur_ids, *weights)

  return scores[:, 0]
